# baseline (speedup 1.0000x reference)
import functools
import math

import jax
import jax.numpy as jnp
from jax import lax
from jax.experimental import pallas as pl
from jax.experimental.pallas import tpu as pltpu
from jax.experimental.pallas import tpu_sc as plsc

D_MODEL = 64
SCALE = math.sqrt(D_MODEL)

NC = 2
NS = 16
NW = NC * NS

NB = 128
NBUF = 4
LANES = 16


def _emb_body(x_hbm, table_hbm, out_hbm, idx_v, bufs, gsems, ssems):
    wid = lax.axis_index("s") * NC + lax.axis_index("c")
    nblk = x_hbm.shape[0] // NW
    blk0 = wid * nblk

    pltpu.sync_copy(x_hbm.at[pl.ds(blk0, nblk)], idx_v)

    def start_gather(b, g):
        pltpu.async_copy(table_hbm.at[idx_v.at[g]], bufs[b], gsems[b])

    def start_store(b, g):
        pltpu.async_copy(bufs[b], out_hbm.at[pl.ds((blk0 + g) * NB, NB)],
                         ssems[b])

    for b in range(NBUF - 1):
        start_gather(b, b)

    def round_body(r):
        for b in range(NBUF):
            g = r * NBUF + b

            pltpu.make_async_copy(table_hbm.at[idx_v.at[g]], bufs[b],
                                  gsems[b]).wait()

            def scale_row(row, _):
                for c in range(D_MODEL // LANES):
                    sl = pl.ds(c * LANES, LANES)
                    bufs[b][row, sl] = bufs[b][row, sl] * SCALE
                return 0

            lax.fori_loop(0, NB, scale_row, 0, unroll=4)

            start_store(b, g)

            bp = (b - 1) % NBUF
            gp = g - 1

            @pl.when(gp >= 0)
            def _():
                pltpu.make_async_copy(
                    bufs[bp], out_hbm.at[pl.ds((blk0 + gp) * NB, NB)],
                    ssems[bp]).wait()

            @pl.when(gp + NBUF < nblk)
            def _():
                start_gather(bp, gp + NBUF)

    pl.loop(0, nblk // NBUF)(round_body)

    bl = (nblk - 1) % NBUF
    pltpu.make_async_copy(bufs[bl],
                          out_hbm.at[pl.ds((blk0 + nblk - 1) * NB, NB)],
                          ssems[bl]).wait()


@jax.jit
def _emb_call(x2d, table):
    n_total = x2d.shape[0] * NB
    mesh = plsc.VectorSubcoreMesh(core_axis_name="c", subcore_axis_name="s",
                                  num_cores=NC, num_subcores=NS)
    nblk_per_w = x2d.shape[0] // NW
    scratch = (
        [pltpu.VMEM((nblk_per_w, NB), jnp.int32)]
        + [[pltpu.VMEM((NB, D_MODEL), jnp.float32) for _ in range(NBUF)]]
        + [[pltpu.SemaphoreType.DMA for _ in range(NBUF)]]
        + [[pltpu.SemaphoreType.DMA for _ in range(NBUF)]]
    )
    kern = pl.kernel(
        _emb_body,
        out_type=jax.ShapeDtypeStruct((n_total, D_MODEL), jnp.float32),
        mesh=mesh,
        scratch_types=scratch,
        compiler_params=pltpu.CompilerParams(use_tc_tiling_on_sc=False),
    )
    return kern(x2d, table)


def kernel(x, table):
    b, s = x.shape
    x2d = x.reshape(-1, NB)
    out = _emb_call(x2d, table)
    return out.reshape(b, s, D_MODEL)
